# baseline (speedup 1.0000x reference)
import functools

import jax
import jax.numpy as jnp
from jax import lax
from jax.experimental import pallas as pl
from jax.experimental.pallas import tpu as pltpu
from jax.experimental.pallas import tpu_sc as plsc

_NUM_CORES = 2
_NUM_SUBCORES = 16
_NUM_WORKERS = _NUM_CORES * _NUM_SUBCORES


def _make_sc_gather(V, E, B):
    assert B % _NUM_WORKERS == 0 and E % 128 == 0
    b_per_w = B // _NUM_WORKERS
    chunk = 128
    assert b_per_w % chunk == 0
    n_chunks = b_per_w // chunk

    mesh = plsc.VectorSubcoreMesh(core_axis_name="c", subcore_axis_name="s")

    @functools.partial(
        pl.kernel,
        mesh=mesh,
        out_type=jax.ShapeDtypeStruct((B, E), jnp.float32),
        scratch_types=[
            pltpu.VMEM((b_per_w,), jnp.int32),
            pltpu.VMEM((2, chunk, E), jnp.float32),
            pltpu.SemaphoreType.DMA,
            pltpu.SemaphoreType.DMA,
            pltpu.SemaphoreType.DMA,
            pltpu.SemaphoreType.DMA,
        ],
    )
    def gather_kernel(table_hbm, idx_hbm, out_hbm, idx_v, rows_v, ga, gb, sa, sb):
        wid = lax.axis_index("s") * _NUM_CORES + lax.axis_index("c")
        base = wid * b_per_w
        pltpu.sync_copy(idx_hbm.at[pl.ds(base, b_per_w)], idx_v)

        def start_gather(c, buf, sem):
            return pltpu.async_copy(
                table_hbm.at[idx_v.at[pl.ds(c * chunk, chunk)]],
                rows_v.at[buf],
                sem,
            )

        def start_scatter(c, buf, sem):
            return pltpu.async_copy(
                rows_v.at[buf], out_hbm.at[pl.ds(base + c * chunk, chunk)], sem
            )

        def pair(p, _):
            c = 2 * p
            g0 = start_gather(c, 0, ga)
            g1 = start_gather(c + 1, 1, gb)
            g0.wait()
            s0 = start_scatter(c, 0, sa)
            g1.wait()
            s1 = start_scatter(c + 1, 1, sb)
            s0.wait()
            s1.wait()
            return 0

        lax.fori_loop(0, n_chunks // 2, pair, 0)

    return gather_kernel


def _head_body(emb_ref, w_ref, b_ref, out_ref):
    emb = emb_ref[...].astype(jnp.bfloat16)
    w = w_ref[...].astype(jnp.bfloat16)
    out = jnp.dot(emb, w, preferred_element_type=jnp.float32) + b_ref[...]
    bm, d = out.shape
    out_ref[...] = out.reshape(bm // 8, 8, d)


def _make_head(B, E, D, bm):
    assert B % bm == 0 and bm % 8 == 0
    return pl.pallas_call(
        _head_body,
        grid=(B // bm,),
        in_specs=[
            pl.BlockSpec((bm, E), lambda i: (i, 0)),
            pl.BlockSpec((E, D), lambda i: (0, 0)),
            pl.BlockSpec((1, D), lambda i: (0, 0)),
        ],
        out_specs=pl.BlockSpec((bm // 8, 8, D), lambda i: (i, 0, 0)),
        out_shape=jax.ShapeDtypeStruct((B // 8, 8, D), jnp.float32),
    )


def kernel(idx, tok_table, pos_table, W, b):
    del pos_table
    V, E = tok_table.shape
    D = W.shape[1]
    Bdim, T = idx.shape
    B = Bdim * T
    Epad = 128

    tok_pad = jnp.pad(tok_table, ((0, 0), (0, Epad - E)))
    w_pad = jnp.pad(W, ((0, Epad - E), (0, 0)))
    flat_idx = idx.reshape(B).astype(jnp.int32)
    emb = _make_sc_gather(V, Epad, B)(tok_pad, flat_idx)
    return _make_head(B, Epad, D, bm=2048)(emb, w_pad, b.reshape(1, D))


# speedup vs baseline: 2.1470x; 2.1470x over previous
import functools

import jax
import jax.numpy as jnp
from jax import lax
from jax.experimental import pallas as pl
from jax.experimental.pallas import tpu as pltpu
from jax.experimental.pallas import tpu_sc as plsc

_NUM_CORES = 2
_NUM_SUBCORES = 16
_NUM_WORKERS = _NUM_CORES * _NUM_SUBCORES


def _make_sc_gather(V, E, B):
    assert B % _NUM_WORKERS == 0 and E % 128 == 0
    b_per_w = B // _NUM_WORKERS
    chunk = 128
    assert b_per_w % chunk == 0
    n_chunks = b_per_w // chunk

    mesh = plsc.VectorSubcoreMesh(core_axis_name="c", subcore_axis_name="s")

    @functools.partial(
        pl.kernel,
        mesh=mesh,
        out_type=jax.ShapeDtypeStruct((B, E), jnp.float32),
        scratch_types=[
            pltpu.VMEM((b_per_w,), jnp.int32),
            pltpu.VMEM((2, chunk, E), jnp.float32),
            pltpu.SemaphoreType.DMA,
            pltpu.SemaphoreType.DMA,
            pltpu.SemaphoreType.DMA,
            pltpu.SemaphoreType.DMA,
        ],
    )
    def gather_kernel(table_hbm, idx_hbm, out_hbm, idx_v, rows_v, ga, gb, sa, sb):
        wid = lax.axis_index("s") * _NUM_CORES + lax.axis_index("c")
        base = wid * b_per_w
        pltpu.sync_copy(idx_hbm.at[pl.ds(base, b_per_w)], idx_v)

        def start_gather(c, buf, sem):
            return pltpu.async_copy(
                table_hbm.at[idx_v.at[pl.ds(c * chunk, chunk)]],
                rows_v.at[buf],
                sem,
            )

        def start_scatter(c, buf, sem):
            return pltpu.async_copy(
                rows_v.at[buf], out_hbm.at[pl.ds(base + c * chunk, chunk)], sem
            )

        def pair(p, _):
            c = 2 * p
            g0 = start_gather(c, 0, ga)
            g1 = start_gather(c + 1, 1, gb)
            g0.wait()
            s0 = start_scatter(c, 0, sa)
            g1.wait()
            s1 = start_scatter(c + 1, 1, sb)
            s0.wait()
            s1.wait()
            return 0

        lax.fori_loop(0, n_chunks // 2, pair, 0)

    return gather_kernel


def _head_body(emb_ref, wt_ref, b_ref, out_ref):
    emb = emb_ref[...].astype(jnp.bfloat16)
    wt = wt_ref[...].astype(jnp.bfloat16)
    out = lax.dot_general(
        wt, emb, (((1,), (1,)), ((), ())),
        preferred_element_type=jnp.float32,
    ) + b_ref[...]
    d, bn = out.shape
    out_ref[...] = out.reshape(1, d, bn)


def _make_head(T, Bd, E, D, bn):
    assert Bd % bn == 0
    nj = Bd // bn
    return pl.pallas_call(
        _head_body,
        grid=(T, nj),
        in_specs=[
            pl.BlockSpec((bn, E), lambda t, j: (t * nj + j, 0)),
            pl.BlockSpec((D, E), lambda t, j: (0, 0)),
            pl.BlockSpec((D, 1), lambda t, j: (0, 0)),
        ],
        out_specs=pl.BlockSpec((1, D, bn), lambda t, j: (t, 0, j)),
        out_shape=jax.ShapeDtypeStruct((T, D, Bd), jnp.float32),
    )


def kernel(idx, tok_table, pos_table, W, b):
    del pos_table
    V, E = tok_table.shape
    D = W.shape[1]
    Bdim, T = idx.shape
    B = Bdim * T
    Epad = 128

    tok_pad = jnp.pad(tok_table, ((0, 0), (0, Epad - E)))
    wt_pad = jnp.pad(W.T, ((0, 0), (0, Epad - E)))
    flat_idx = idx.T.reshape(B).astype(jnp.int32)
    emb = _make_sc_gather(V, Epad, B)(tok_pad, flat_idx)
    out_t = _make_head(T, Bdim, Epad, D, bn=1024)(emb, wt_pad, b.reshape(D, 1))
    return jnp.transpose(out_t, (2, 0, 1))


# speedup vs baseline: 2.3511x; 1.0950x over previous
import functools

import jax
import jax.numpy as jnp
from jax import lax
from jax.experimental import pallas as pl
from jax.experimental.pallas import tpu as pltpu
from jax.experimental.pallas import tpu_sc as plsc

_NUM_CORES = 2
_NUM_SUBCORES = 16
_NUM_WORKERS = _NUM_CORES * _NUM_SUBCORES


def _make_sc_gather(V, E, B):
    assert B % _NUM_WORKERS == 0 and E % 128 == 0
    b_per_w = B // _NUM_WORKERS
    chunk = 128
    assert b_per_w % chunk == 0
    n_chunks = b_per_w // chunk
    nbuf = 4
    assert n_chunks >= nbuf

    mesh = plsc.VectorSubcoreMesh(core_axis_name="c", subcore_axis_name="s")

    @functools.partial(
        pl.kernel,
        mesh=mesh,
        out_type=jax.ShapeDtypeStruct((B, E), jnp.float32),
        scratch_types=[
            pltpu.VMEM((b_per_w,), jnp.int32),
            pltpu.VMEM((nbuf, chunk, E), jnp.float32),
            [pltpu.SemaphoreType.DMA] * nbuf,
            [pltpu.SemaphoreType.DMA] * nbuf,
        ],
    )
    def gather_kernel(table_hbm, idx_hbm, out_hbm, idx_v, rows_v, gsems, ssems):
        wid = lax.axis_index("s") * _NUM_CORES + lax.axis_index("c")
        base = wid * b_per_w
        pltpu.sync_copy(idx_hbm.at[pl.ds(base, b_per_w)], idx_v)

        def start_gather(c, buf):
            return pltpu.async_copy(
                table_hbm.at[idx_v.at[pl.ds(c * chunk, chunk)]],
                rows_v.at[buf],
                gsems[buf],
            )

        def start_scatter(c, buf):
            return pltpu.async_copy(
                rows_v.at[buf],
                out_hbm.at[pl.ds(base + c * chunk, chunk)],
                ssems[buf],
            )

        gathers = [start_gather(c, c) for c in range(nbuf)]
        scatters = [None] * n_chunks
        for c in range(n_chunks):
            b = c % nbuf
            gathers[b].wait()
            scatters[c] = start_scatter(c, b)
            if c + nbuf < n_chunks:
                scatters[c].wait()
                gathers[b] = start_gather(c + nbuf, b)
        for c in range(n_chunks - nbuf, n_chunks):
            scatters[c].wait()

    return gather_kernel


def _head_body(emb_ref, wt_ref, b_ref, out_ref):
    emb = emb_ref[...].astype(jnp.bfloat16)
    wt = wt_ref[...].astype(jnp.bfloat16)
    out = lax.dot_general(
        wt, emb, (((1,), (1,)), ((), ())),
        preferred_element_type=jnp.float32,
    ) + b_ref[...]
    d, bn = out.shape
    out_ref[...] = out.reshape(1, d, bn)


def _make_head(T, Bd, E, D, bn):
    assert Bd % bn == 0
    nj = Bd // bn
    return pl.pallas_call(
        _head_body,
        grid=(T, nj),
        in_specs=[
            pl.BlockSpec((bn, E), lambda t, j: (t * nj + j, 0)),
            pl.BlockSpec((D, E), lambda t, j: (0, 0)),
            pl.BlockSpec((D, 1), lambda t, j: (0, 0)),
        ],
        out_specs=pl.BlockSpec((1, D, bn), lambda t, j: (t, 0, j)),
        out_shape=jax.ShapeDtypeStruct((T, D, Bd), jnp.float32),
    )


def kernel(idx, tok_table, pos_table, W, b):
    del pos_table
    V, E = tok_table.shape
    D = W.shape[1]
    Bdim, T = idx.shape
    B = Bdim * T
    Epad = 128

    tok_pad = jnp.pad(tok_table, ((0, 0), (0, Epad - E)))
    wt_pad = jnp.pad(W.T, ((0, 0), (0, Epad - E)))
    flat_idx = idx.T.reshape(B).astype(jnp.int32)
    emb = _make_sc_gather(V, Epad, B)(tok_pad, flat_idx)
    out_t = _make_head(T, Bdim, Epad, D, bn=2048)(emb, wt_pad, b.reshape(D, 1))
    return jnp.transpose(out_t, (2, 0, 1))


# speedup vs baseline: 2.5467x; 1.0832x over previous
import functools

import jax
import jax.numpy as jnp
from jax import lax
from jax.experimental import pallas as pl
from jax.experimental.pallas import tpu as pltpu
from jax.experimental.pallas import tpu_sc as plsc

_NUM_CORES = 2
_NUM_SUBCORES = 16
_NUM_WORKERS = _NUM_CORES * _NUM_SUBCORES


def _make_sc_gather(V, E, B):
    assert B % _NUM_WORKERS == 0
    b_per_w = B // _NUM_WORKERS
    chunk = 128
    assert b_per_w % chunk == 0
    n_chunks = b_per_w // chunk
    nbuf = 4
    assert n_chunks >= nbuf

    mesh = plsc.VectorSubcoreMesh(core_axis_name="c", subcore_axis_name="s")

    @functools.partial(
        pl.kernel,
        mesh=mesh,
        compiler_params=pltpu.CompilerParams(use_tc_tiling_on_sc=False),
        out_type=jax.ShapeDtypeStruct((B, E), jnp.bfloat16),
        scratch_types=[
            pltpu.VMEM((b_per_w,), jnp.int32),
            pltpu.VMEM((nbuf, chunk, E), jnp.bfloat16),
            [pltpu.SemaphoreType.DMA] * nbuf,
            [pltpu.SemaphoreType.DMA] * nbuf,
        ],
    )
    def gather_kernel(table_hbm, idx_hbm, out_hbm, idx_v, rows_v, gsems, ssems):
        wid = lax.axis_index("s") * _NUM_CORES + lax.axis_index("c")
        base = wid * b_per_w
        pltpu.sync_copy(idx_hbm.at[pl.ds(base, b_per_w)], idx_v)

        def start_gather(c, buf):
            return pltpu.async_copy(
                table_hbm.at[idx_v.at[pl.ds(c * chunk, chunk)]],
                rows_v.at[buf],
                gsems[buf],
            )

        def start_scatter(c, buf):
            return pltpu.async_copy(
                rows_v.at[buf],
                out_hbm.at[pl.ds(base + c * chunk, chunk)],
                ssems[buf],
            )

        gathers = [start_gather(c, c) for c in range(nbuf)]
        scatters = [None] * n_chunks
        for c in range(n_chunks):
            b = c % nbuf
            gathers[b].wait()
            scatters[c] = start_scatter(c, b)
            if c + nbuf < n_chunks:
                scatters[c].wait()
                gathers[b] = start_gather(c + nbuf, b)
        for c in range(n_chunks - nbuf, n_chunks):
            scatters[c].wait()

    return gather_kernel


def _head_body(emb_ref, wt_ref, b_ref, out_ref):
    out = lax.dot_general(
        wt_ref[...], emb_ref[...], (((1,), (1,)), ((), ())),
        preferred_element_type=jnp.float32,
    ) + b_ref[...]
    d, bn = out.shape
    out_ref[...] = out.reshape(1, d, bn)


def _make_head(T, Bd, E, D, bn):
    assert Bd % bn == 0
    nj = Bd // bn
    return pl.pallas_call(
        _head_body,
        grid=(T, nj),
        in_specs=[
            pl.BlockSpec((bn, E), lambda t, j: (t * nj + j, 0)),
            pl.BlockSpec((D, E), lambda t, j: (0, 0)),
            pl.BlockSpec((D, 1), lambda t, j: (0, 0)),
        ],
        out_specs=pl.BlockSpec((1, D, bn), lambda t, j: (t, 0, j)),
        out_shape=jax.ShapeDtypeStruct((T, D, Bd), jnp.float32),
    )


def kernel(idx, tok_table, pos_table, W, b):
    del pos_table
    V, E = tok_table.shape
    D = W.shape[1]
    Bdim, T = idx.shape
    B = Bdim * T

    tok_bf = tok_table.astype(jnp.bfloat16)
    wt_bf = W.T.astype(jnp.bfloat16)
    flat_idx = idx.T.reshape(B).astype(jnp.int32)
    emb = _make_sc_gather(V, E, B)(tok_bf, flat_idx)
    out_t = _make_head(T, Bdim, E, D, bn=2048)(emb, wt_bf, b.reshape(D, 1))
    return jnp.transpose(out_t, (2, 0, 1))


# speedup vs baseline: 2.6786x; 1.0518x over previous
import functools

import jax
import jax.numpy as jnp
from jax import lax
from jax.experimental import pallas as pl
from jax.experimental.pallas import tpu as pltpu
from jax.experimental.pallas import tpu_sc as plsc

_NUM_CORES = 2
_NUM_SUBCORES = 16
_NUM_WORKERS = _NUM_CORES * _NUM_SUBCORES


def _make_sc_gather(V, E, B):
    assert B % _NUM_WORKERS == 0
    b_per_w = B // _NUM_WORKERS
    chunk = 128
    assert b_per_w % chunk == 0
    n_chunks = b_per_w // chunk
    nbuf = 4
    assert n_chunks >= nbuf

    mesh = plsc.VectorSubcoreMesh(core_axis_name="c", subcore_axis_name="s")

    @functools.partial(
        pl.kernel,
        mesh=mesh,
        compiler_params=pltpu.CompilerParams(use_tc_tiling_on_sc=False),
        out_type=jax.ShapeDtypeStruct((B, E), jnp.float32),
        scratch_types=[
            pltpu.VMEM((b_per_w,), jnp.int32),
            pltpu.VMEM((nbuf, chunk, E), jnp.float32),
            [pltpu.SemaphoreType.DMA] * nbuf,
            [pltpu.SemaphoreType.DMA] * nbuf,
        ],
    )
    def gather_kernel(table_hbm, idx_hbm, out_hbm, idx_v, rows_v, gsems, ssems):
        wid = lax.axis_index("s") * _NUM_CORES + lax.axis_index("c")
        base = wid * b_per_w
        pltpu.sync_copy(idx_hbm.at[pl.ds(base, b_per_w)], idx_v)

        def start_gather(c, buf):
            return pltpu.async_copy(
                table_hbm.at[idx_v.at[pl.ds(c * chunk, chunk)]],
                rows_v.at[buf],
                gsems[buf],
            )

        def start_scatter(c, buf):
            return pltpu.async_copy(
                rows_v.at[buf],
                out_hbm.at[pl.ds(base + c * chunk, chunk)],
                ssems[buf],
            )

        gathers = [start_gather(c, c) for c in range(nbuf)]
        scatters = [None] * n_chunks
        for c in range(n_chunks):
            b = c % nbuf
            gathers[b].wait()
            scatters[c] = start_scatter(c, b)
            if c + nbuf < n_chunks:
                scatters[c].wait()
                gathers[b] = start_gather(c + nbuf, b)
        for c in range(n_chunks - nbuf, n_chunks):
            scatters[c].wait()

    return gather_kernel


def _head_body(emb_ref, wt_ref, b_ref, out_ref):
    g, _ = emb_ref.shape
    wt = wt_ref[...]
    bias = b_ref[...]
    for p in range(4):
        vp = emb_ref[:, 32 * p:32 * (p + 1)].astype(jnp.bfloat16)
        out = lax.dot_general(
            wt, vp, (((1,), (1,)), ((), ())),
            preferred_element_type=jnp.float32,
        ) + bias
        out_ref[0, :, g * p:g * (p + 1)] = out


def _make_head(T, Bd, E, D, bn):
    assert Bd % bn == 0 and bn % 4 == 0
    nj = Bd // bn
    gb = bn // 4
    return pl.pallas_call(
        _head_body,
        grid=(T, nj),
        in_specs=[
            pl.BlockSpec((gb, 4 * E), lambda t, j: (t * nj + j, 0)),
            pl.BlockSpec((D, E), lambda t, j: (0, 0)),
            pl.BlockSpec((D, 1), lambda t, j: (0, 0)),
        ],
        out_specs=pl.BlockSpec((1, D, bn), lambda t, j: (t, 0, j)),
        out_shape=jax.ShapeDtypeStruct((T, D, Bd), jnp.float32),
    )


def kernel(idx, tok_table, pos_table, W, b):
    del pos_table
    V, E = tok_table.shape
    D = W.shape[1]
    Bdim, T = idx.shape
    B = Bdim * T
    bn = 2048
    nj = Bdim // bn

    wt_bf = W.T.astype(jnp.bfloat16)
    idx_t = idx.T.reshape(T, nj, 4, bn // 4)
    flat_idx = jnp.transpose(idx_t, (0, 1, 3, 2)).reshape(B).astype(jnp.int32)
    emb = _make_sc_gather(V, E, B)(tok_table, flat_idx)
    emb_p = emb.reshape(B // 4, 4 * E)
    out_t = _make_head(T, Bdim, E, D, bn)(emb_p, wt_bf, b.reshape(D, 1))
    return jnp.transpose(out_t, (2, 0, 1))


# speedup vs baseline: 2.8008x; 1.0456x over previous
import functools

import jax
import jax.numpy as jnp
from jax import lax
from jax.experimental import pallas as pl
from jax.experimental.pallas import tpu as pltpu
from jax.experimental.pallas import tpu_sc as plsc

_NUM_CORES = 2
_NUM_SUBCORES = 16
_NUM_WORKERS = _NUM_CORES * _NUM_SUBCORES


def _make_sc_gather(V, E, B):
    assert B % _NUM_WORKERS == 0
    b_per_w = B // _NUM_WORKERS
    chunk = 128
    assert b_per_w % chunk == 0
    n_chunks = b_per_w // chunk
    nbuf = 4
    assert n_chunks >= nbuf

    mesh = plsc.VectorSubcoreMesh(core_axis_name="c", subcore_axis_name="s")

    quarter = b_per_w // 4

    @functools.partial(
        pl.kernel,
        mesh=mesh,
        compiler_params=pltpu.CompilerParams(
            use_tc_tiling_on_sc=False, needs_layout_passes=False
        ),
        out_type=jax.ShapeDtypeStruct((B, E), jnp.float32),
        scratch_types=[
            pltpu.VMEM((b_per_w,), jnp.int32),
            pltpu.VMEM((4, quarter), jnp.int32),
            pltpu.VMEM((nbuf, chunk, E), jnp.float32),
            [pltpu.SemaphoreType.DMA] * nbuf,
            [pltpu.SemaphoreType.DMA] * nbuf,
        ],
    )
    def gather_kernel(
        table_hbm, idx_hbm, out_hbm, idx_v, idx_raw, rows_v, gsems, ssems
    ):
        wid = lax.axis_index("s") * _NUM_CORES + lax.axis_index("c")
        base = wid * b_per_w
        blk_base = (base // (8 * quarter)) * (8 * quarter)
        i0 = (base - blk_base) // 4
        for p in range(4):
            off = pl.multiple_of(blk_base + 2 * quarter * p + i0, 8)
            pltpu.sync_copy(
                idx_hbm.at[pl.ds(off, quarter)],
                idx_raw.at[p],
            )
        lanes = lax.broadcasted_iota(jnp.int32, (16,), 0)
        for p in range(4):
            for k in range(quarter // 16):
                vals = idx_raw[p, pl.ds(16 * k, 16)]
                tgt = (lanes + 16 * k) * 4 + p
                plsc.store_scatter(idx_v, [tgt], vals)

        def start_gather(c, buf):
            return pltpu.async_copy(
                table_hbm.at[idx_v.at[pl.ds(c * chunk, chunk)]],
                rows_v.at[buf],
                gsems[buf],
            )

        def start_scatter(c, buf):
            return pltpu.async_copy(
                rows_v.at[buf],
                out_hbm.at[pl.ds(base + c * chunk, chunk)],
                ssems[buf],
            )

        gathers = [start_gather(c, c) for c in range(nbuf)]
        scatters = [None] * n_chunks
        for c in range(n_chunks):
            b = c % nbuf
            gathers[b].wait()
            scatters[c] = start_scatter(c, b)
            if c + nbuf < n_chunks:
                scatters[c].wait()
                gathers[b] = start_gather(c + nbuf, b)
        for c in range(n_chunks - nbuf, n_chunks):
            scatters[c].wait()

    return gather_kernel


def _head_body(emb_ref, wt_ref, b_ref, out_ref):
    g, _ = emb_ref.shape
    wt = wt_ref[...]
    bias = b_ref[...]
    for p in range(4):
        vp = emb_ref[:, 32 * p:32 * (p + 1)].astype(jnp.bfloat16)
        out = lax.dot_general(
            wt, vp, (((1,), (1,)), ((), ())),
            preferred_element_type=jnp.float32,
        ) + bias
        out_ref[0, :, g * p:g * (p + 1)] = out


def _make_head(T, Bd, E, D, bn):
    assert Bd % bn == 0 and bn % 4 == 0
    nj = Bd // bn
    gb = bn // 4
    return pl.pallas_call(
        _head_body,
        grid=(T, nj),
        in_specs=[
            pl.BlockSpec((gb, 4 * E), lambda t, j: (t * nj + j, 0)),
            pl.BlockSpec((D, E), lambda t, j: (0, 0)),
            pl.BlockSpec((D, 1), lambda t, j: (0, 0)),
        ],
        out_specs=pl.BlockSpec((1, D, bn), lambda t, j: (t, 0, j)),
        out_shape=jax.ShapeDtypeStruct((T, D, Bd), jnp.float32),
    )


def kernel(idx, tok_table, pos_table, W, b):
    del pos_table
    V, E = tok_table.shape
    D = W.shape[1]
    Bdim, T = idx.shape
    B = Bdim * T
    bn = 2048
    nj = Bdim // bn

    wt_bf = W.T.astype(jnp.bfloat16)
    flat_idx = idx.T.reshape(B).astype(jnp.int32)
    emb = _make_sc_gather(V, E, B)(tok_table, flat_idx)
    emb_p = emb.reshape(B // 4, 4 * E)
    out_t = _make_head(T, Bdim, E, D, bn)(emb_p, wt_bf, b.reshape(D, 1))
    return jnp.transpose(out_t, (2, 0, 1))
